# Initial kernel scaffold; baseline (speedup 1.0000x reference)
import jax
import jax.numpy as jnp
from jax.experimental import pallas as pl


def kernel(x, edge_attr, edge_index, batch, target_index, target_class, params):
    raise NotImplementedError("write your pallas kernel here")


# speedup vs baseline: 1.0001x; 1.0001x over previous
import functools

import jax
import jax.numpy as jnp
from jax import lax
from jax.experimental import pallas as pl
from jax.experimental.pallas import tpu as pltpu
from jax.experimental.pallas import tpu_sc as plsc

N = 10000
E = 160000
F_NODE = 128
F_EDGE = 16
D = 32
B = 128
T = 4096
N_OUT = 8
STEPS = 3

NC = 2
NS = 16
NW = NC * NS
CH = 128
ROWS_W = 40
EP = NW * ROWS_W * CH
NP = 10016
RT = NP // NS

_f32 = jnp.float32


def _dgt(a, b, ca, cb, prec=lax.Precision.HIGHEST):
    return lax.dot_general(a, b, (((ca,), (cb,)), ((), ())),
                           preferred_element_type=_f32, precision=prec)


def _node_pre_body(x_ref, w1, b1, g1, bb1, w2, b2, g2, bb2, out_ref):
    x = x_ref[...]
    t = _dgt(x, w1[...], 1, 0) + b1[...]
    m = jnp.mean(t, 0, keepdims=True)
    v = jnp.mean((t - m) ** 2, 0, keepdims=True)
    t = jax.nn.relu(g1[...] * (t - m) / jnp.sqrt(v + 1e-5) + bb1[...])
    t2 = _dgt(t, w2[...], 1, 0) + b2[...]
    m2 = jnp.mean(t2, 0, keepdims=True)
    v2 = jnp.mean((t2 - m2) ** 2, 0, keepdims=True)
    out_ref[...] = jax.nn.relu(g2[...] * (t2 - m2) / jnp.sqrt(v2 + 1e-5) + bb2[...])


def _node_pre(x, p):
    return pl.pallas_call(
        _node_pre_body,
        out_shape=jax.ShapeDtypeStruct((N, D), _f32),
    )(x, p['W1'], p['b1'].reshape(1, D), p['bn1g'].reshape(1, D),
      p['bn1b'].reshape(1, D), p['W2'], p['b2'].reshape(1, D),
      p['bn2g'].reshape(1, D), p['bn2b'].reshape(1, D))


GP = 8
GR = E // GP
GRP = EP // GP


def _stats_body(a_ref, k_ref, cs_ref):
    i = pl.program_id(0)
    a = a_ref[...]
    kt = _dgt(a, a, 0, 0)
    cst = jnp.sum(a, 0, keepdims=True)

    @pl.when(i == 0)
    def _():
        k_ref[...] = kt
        cs_ref[...] = cst

    @pl.when(i > 0)
    def _():
        k_ref[...] += kt
        cs_ref[...] += cst


def _stats(arr, tr):
    r, l = arr.shape
    return pl.pallas_call(
        _stats_body,
        grid=(r // tr,),
        in_specs=[pl.BlockSpec((tr, l), lambda i: (i, 0))],
        out_specs=(pl.BlockSpec((l, l), lambda i: (0, 0)),
                   pl.BlockSpec((1, l), lambda i: (0, 0))),
        out_shape=(jax.ShapeDtypeStruct((l, l), _f32),
                   jax.ShapeDtypeStruct((1, l), _f32)),
    )(arr)


def _fold1_body(k_ref, cs_ref, we1, be1, g1, bb1, wbd_ref, c1t_ref):
    k1 = k_ref[...]
    cs = cs_ref[...]
    s = sum(k1[i * F_EDGE:(i + 1) * F_EDGE, i * F_EDGE:(i + 1) * F_EDGE]
            for i in range(GP))
    sm = sum(cs[:, i * F_EDGE:(i + 1) * F_EDGE] for i in range(GP))
    mu = sm / E
    m1 = _dgt(mu, we1[...], 1, 0) + be1[...]
    cov = s / E - _dgt(mu, mu, 0, 0)
    v1 = jnp.sum(we1[...] * _dgt(cov, we1[...], 1, 0), 0, keepdims=True)
    a1 = g1[...] / jnp.sqrt(v1 + 1e-5)
    w1f = we1[...] * a1
    c1f = a1 * (be1[...] - m1) + bb1[...]
    wbd_ref[...] = jnp.zeros((GP * F_EDGE, GP * D), _f32)
    for i in range(GP):
        wbd_ref[pl.ds(i * F_EDGE, F_EDGE), pl.ds(i * D, D)] = w1f
        c1t_ref[:, pl.ds(i * D, D)] = c1f


def _fold1(s_ea, sum_ea, p):
    return pl.pallas_call(
        _fold1_body,
        out_shape=(jax.ShapeDtypeStruct((GP * F_EDGE, GP * D), _f32),
                   jax.ShapeDtypeStruct((1, GP * D), _f32)),
    )(s_ea, sum_ea, p['We1'], p['be1'].reshape(1, D),
      p['bne1g'].reshape(1, D), p['bne1b'].reshape(1, D))


EBLK_G = GR // 10
PBLK_G = GRP // 10


def _e1_body(g_ref, wbd, c1t, e1_ref):
    g = g_ref[...]
    h = jax.nn.relu(_dgt(g, wbd[...], 1, 0) + c1t[...])
    e1_ref[pl.ds(0, EBLK_G), :] = h
    e1_ref[pl.ds(EBLK_G, PBLK_G - EBLK_G), :] = (
        jnp.zeros((PBLK_G - EBLK_G, GP * D), _f32))


def _e1_compute(gview, wbd, c1t):
    return pl.pallas_call(
        _e1_body,
        grid=(10,),
        in_specs=[pl.BlockSpec((EBLK_G, GP * F_EDGE), lambda i: (i, 0)),
                  pl.BlockSpec((GP * F_EDGE, GP * D), lambda i: (0, 0)),
                  pl.BlockSpec((1, GP * D), lambda i: (0, 0))],
        out_specs=pl.BlockSpec((PBLK_G, GP * D), lambda i: (i, 0)),
        out_shape=jax.ShapeDtypeStruct((GRP, GP * D), _f32),
    )(gview, wbd, c1t)


def _fold2_body(k_ref, cs_ref, we2, be2, g2, bb2, w2f_ref, c2f_ref):
    k2 = k_ref[...]
    cs = cs_ref[...]
    s1 = sum(k2[i * D:(i + 1) * D, i * D:(i + 1) * D] for i in range(GP))
    sum1 = sum(cs[:, i * D:(i + 1) * D] for i in range(GP))
    mu1 = sum1 / E
    m2 = _dgt(mu1, we2[...], 1, 0) + be2[...]
    cov1 = s1 / E - _dgt(mu1, mu1, 0, 0)
    cw = _dgt(cov1, we2[...], 1, 0)
    v2 = jnp.sum(we2[...] * cw, 0, keepdims=True)
    a2 = g2[...] / jnp.sqrt(v2 + 1e-5)
    w2f_ref[...] = we2[...] * a2
    c2f_ref[...] = a2 * (be2[...] - m2) + bb2[...]


def _fold2(s1, sum1, p):
    return pl.pallas_call(
        _fold2_body,
        out_shape=(jax.ShapeDtypeStruct((D, D * D), _f32),
                   jax.ShapeDtypeStruct((1, D * D), _f32)),
    )(s1, sum1, p['We2'], p['be2'].reshape(1, D * D),
      p['bne2g'].reshape(1, D * D), p['bne2b'].reshape(1, D * D))


def _edge_pre(edge_attr, p):
    gview = edge_attr.reshape(GR, GP * F_EDGE)
    k1, cs1 = _stats(gview, 2000)
    wbd, c1t = _fold1(k1, cs1, p)
    e1q = _e1_compute(gview, wbd, c1t)
    k2, cs2 = _stats(e1q, 2048)
    w2f, c2f = _fold2(k2, cs2, p)
    return e1q.reshape(EP, D), w2f, c2f


TE = 2048


def _msg_body(xj_ref, e1_ref, m_ref, c_ref, out_ref):
    xj = xj_ref[...]
    e1 = e1_ref[...]
    o = jnp.concatenate([xj[:, d:d + 1] * e1 for d in range(D)], 1)
    acc = _dgt(o, m_ref[...], 1, 0)
    out_ref[...] = acc + _dgt(xj, c_ref[...], 1, 0)


def _msg(xjp, e1p, mmat, cmat):
    ge = EP // TE
    return pl.pallas_call(
        _msg_body,
        grid=(ge,),
        in_specs=[pl.BlockSpec((TE, D), lambda i: (i, 0)),
                  pl.BlockSpec((TE, D), lambda i: (i, 0)),
                  pl.BlockSpec((D * D, D), lambda i: (0, 0)),
                  pl.BlockSpec((D, D), lambda i: (0, 0))],
        out_specs=pl.BlockSpec((TE, D), lambda i: (i, 0)),
        out_shape=jax.ShapeDtypeStruct((EP, D), _f32),
    )(xjp, e1p, mmat, cmat)


def _gru_body(out_ref, h_ref, a0_ref, a1_ref, d0_ref, d1_ref,
              wroot, broot, wx, bx, wh, bh, lng, lnb, new_ref):
    deg = jnp.maximum(d0_ref[...] + d1_ref[...], 1.0)
    agg = (a0_ref[...] + a1_ref[...]) / deg
    out = out_ref[...]
    h = h_ref[...]
    m = jax.nn.relu(_dgt(out, wroot[...], 1, 0) + broot[...] + agg)
    gi = _dgt(m, wx[...], 1, 0) + bx[...]
    gh = _dgt(h, wh[...], 1, 0) + bh[...]
    r = jax.nn.sigmoid(gi[:, 0:D] + gh[:, 0:D])
    z = jax.nn.sigmoid(gi[:, D:2 * D] + gh[:, D:2 * D])
    ng = jnp.tanh(gi[:, 2 * D:3 * D] + r * gh[:, 2 * D:3 * D])
    hn = (1.0 - z) * ng + z * h
    mh = jnp.mean(hn, -1, keepdims=True)
    vh = jnp.mean((hn - mh) ** 2, -1, keepdims=True)
    new_ref[...] = lng[...] * (hn - mh) / jnp.sqrt(vh + 1e-5) + lnb[...]


def _gru(out, h, a0, a1, d0, d1, p):
    trn = 2000
    row = lambda i: (i, 0)
    full = lambda i: (0, 0)
    return pl.pallas_call(
        _gru_body,
        grid=(N // trn,),
        in_specs=[pl.BlockSpec((trn, D), row)] * 4 +
                 [pl.BlockSpec((trn, 1), row)] * 2 +
                 [pl.BlockSpec((D, D), full), pl.BlockSpec((1, D), full),
                  pl.BlockSpec((D, 3 * D), full), pl.BlockSpec((1, 3 * D), full),
                  pl.BlockSpec((D, 3 * D), full), pl.BlockSpec((1, 3 * D), full),
                  pl.BlockSpec((1, D), full), pl.BlockSpec((1, D), full)],
        out_specs=pl.BlockSpec((trn, D), row),
        out_shape=jax.ShapeDtypeStruct((N, D), _f32),
    )(out, h, a0, a1, d0, d1, p['Wroot'], p['broot'].reshape(1, D),
      p['Wx'], p['bx'].reshape(1, 3 * D), p['Wh'], p['bh'].reshape(1, 3 * D),
      p['lng'].reshape(1, D), p['lnb'].reshape(1, D))


def _ohdot(oh, x, ca, cb):
    hi = x.astype(jnp.bfloat16).astype(_f32)
    lo = x - hi
    return (_dgt(oh, hi, ca, cb, lax.Precision.DEFAULT) +
            _dgt(oh, lo, ca, cb, lax.Precision.DEFAULT))


def _s2s_body(out_ref, b_ref, wxl, whl, bl, tab_ref):
    out = out_ref[...]
    bt = b_ref[...]
    oh = (bt == lax.broadcasted_iota(jnp.int32, (N, B), 1)).astype(_f32)
    qs = jnp.zeros((B, 2 * D), _f32)
    hh = jnp.zeros((B, D), _f32)
    cc = jnp.zeros((B, D), _f32)
    for _ in range(STEPS):
        g = _dgt(qs, wxl[...], 1, 0) + _dgt(hh, whl[...], 1, 0) + bl[...]
        i_ = jax.nn.sigmoid(g[:, 0:D])
        f_ = jax.nn.sigmoid(g[:, D:2 * D])
        gc = jnp.tanh(g[:, 2 * D:3 * D])
        o_ = jax.nn.sigmoid(g[:, 3 * D:4 * D])
        cc = f_ * cc + i_ * gc
        hh = o_ * jnp.tanh(cc)
        ener_all = lax.dot_general(
            out, hh, (((1,), (1,)), ((), ())),
            preferred_element_type=_f32,
            precision=lax.Precision.HIGHEST)
        ener = jnp.sum(oh * ener_all, 1, keepdims=True)
        emax = jnp.max(jnp.where(oh > 0, ener_all, -1e30), 0, keepdims=True)
        epn = jnp.sum(oh * emax, 1, keepdims=True)
        ee = jnp.exp(ener - epn)
        denom = _ohdot(oh, ee, 0, 0)
        dpn = _ohdot(oh, denom, 1, 0)
        a = ee / (dpn + 1e-16)
        rvec = _ohdot(oh, a * out, 0, 0)
        qs = jnp.concatenate([hh, rvec], 1)
    tab_ref[:, 0:D] = out
    tab_ref[:, D:3 * D] = _ohdot(oh, qs, 1, 0)


def _s2s(out, batch2d, p):
    return pl.pallas_call(
        _s2s_body,
        out_shape=jax.ShapeDtypeStruct((N, 3 * D), _f32),
        compiler_params=pltpu.CompilerParams(
            vmem_limit_bytes=100 * 1024 * 1024),
    )(out, batch2d, p['Wxl'], p['Whl'], p['bl'].reshape(1, 4 * D))


def _readout_body(g0_ref, g1_ref, tc_ref, p1, pb1, l1g, l1b,
                  p2, pb2, l2g, l2b, p3, pb3, out_ref):
    g0 = g0_ref[...]
    g1 = g1_ref[...]
    zc = jnp.concatenate([g0[:, 0:D], g1[:, 0:D], g0[:, D:3 * D]], 1)

    def ln_relu(z, g, b):
        m = jnp.mean(z, -1, keepdims=True)
        v = jnp.mean((z - m) ** 2, -1, keepdims=True)
        return jax.nn.relu(g[...] * (z - m) / jnp.sqrt(v + 1e-5) + b[...])

    z1 = ln_relu(_dgt(zc, p1[...], 1, 0) + pb1[...], l1g, l1b)
    z2 = ln_relu(_dgt(z1, p2[...], 1, 0) + pb2[...], l2g, l2b)
    pred = _dgt(z2, p3[...], 1, 0) + pb3[...]
    oh = (tc_ref[...] == lax.broadcasted_iota(jnp.int32, (T, N_OUT), 1))
    out_ref[...] = jnp.sum(pred * oh.astype(_f32), 1, keepdims=True)


def _readout(g0, g1, tc2d, p):
    return pl.pallas_call(
        _readout_body,
        out_shape=jax.ShapeDtypeStruct((T, 1), _f32),
    )(g0, g1, tc2d, p['P1'], p['pb1'].reshape(1, 4 * D),
      p['ln1g'].reshape(1, 4 * D), p['ln1b'].reshape(1, 4 * D),
      p['P2'], p['pb2'].reshape(1, 4 * D), p['ln2g'].reshape(1, 4 * D),
      p['ln2b'].reshape(1, 4 * D), p['P3'], p['pb3'].reshape(1, N_OUT))


@functools.cache
def _mesh():
    return plsc.VectorSubcoreMesh(core_axis_name="c", subcore_axis_name="s",
                                  num_cores=NC, num_subcores=NS)


def _wid():
    return lax.axis_index("s") * NC + lax.axis_index("c")


@functools.partial(jax.jit, static_argnums=(2, 3))
def _sc_gather(table, idx3, w, r):
    @functools.partial(
        pl.kernel,
        out_type=jax.ShapeDtypeStruct((NW * r * CH, w), _f32),
        mesh=_mesh(),
        compiler_params=pltpu.CompilerParams(use_tc_tiling_on_sc=False),
        scratch_types=[pltpu.VMEM((r, CH), jnp.int32),
                       pltpu.VMEM((CH, w), _f32),
                       pltpu.SemaphoreType.DMA],
    )
    def k(table_hbm, idx_hbm, out_hbm, idx_v, buf, sem):
        wid = _wid()
        pltpu.sync_copy(idx_hbm.at[wid], idx_v)

        def body(j, _):
            base = (wid * r + j) * CH
            pltpu.async_copy(table_hbm.at[idx_v.at[j]], buf, sem).wait()
            pltpu.sync_copy(buf, out_hbm.at[pl.ds(base, CH)])
            return 0

        lax.fori_loop(0, r, body, 0)

    return k(table, idx3)


def _zero_fill(ref, rows, w):
    def zb(i, _):
        for c in range(w // 16):
            ref[i, pl.ds(c * 16, 16)] = jnp.zeros((16,), _f32)
        return 0
    lax.fori_loop(0, rows, zb, 0)


@jax.jit
def _sc_scatter(vals, dst3):
    @functools.partial(
        pl.kernel,
        out_type=jax.ShapeDtypeStruct((NC, NP, D), _f32),
        mesh=_mesh(),
        compiler_params=pltpu.CompilerParams(use_tc_tiling_on_sc=False),
        scratch_types=[pltpu.VMEM((ROWS_W, CH), jnp.int32),
                       pltpu.VMEM((CH, D), _f32),
                       pltpu.VMEM((RT, D), _f32),
                       pltpu.VMEM_SHARED((NP, D), _f32)],
    )
    def k(vals_hbm, dst_hbm, aggs_hbm, idx_v, row_v, zbuf, agg_sh):
        cid = lax.axis_index("c")
        sid = lax.axis_index("s")
        wid = sid * NC + cid
        _zero_fill(zbuf, RT, D)
        pltpu.sync_copy(zbuf, agg_sh.at[pl.ds(sid * RT, RT)])
        plsc.subcore_barrier()
        pltpu.sync_copy(dst_hbm.at[wid], idx_v)

        def body(j, _):
            base = (wid * ROWS_W + j) * CH
            pltpu.sync_copy(vals_hbm.at[pl.ds(base, CH)], row_v)
            pltpu.sync_copy(row_v, agg_sh.at[idx_v.at[j]], add=True)
            return 0

        lax.fori_loop(0, ROWS_W, body, 0)
        plsc.subcore_barrier()
        pltpu.sync_copy(agg_sh.at[pl.ds(sid * RT, RT)], zbuf)
        pltpu.sync_copy(zbuf, aggs_hbm.at[cid, pl.ds(sid * RT, RT)])

    return k(vals, dst3)


@jax.jit
def _sc_degree(dst3):
    w = 16

    @functools.partial(
        pl.kernel,
        out_type=jax.ShapeDtypeStruct((NC, NP, w), _f32),
        mesh=_mesh(),
        compiler_params=pltpu.CompilerParams(use_tc_tiling_on_sc=False),
        scratch_types=[pltpu.VMEM((ROWS_W, CH), jnp.int32),
                       pltpu.VMEM((CH, w), _f32),
                       pltpu.VMEM((RT, w), _f32),
                       pltpu.VMEM_SHARED((NP, w), _f32)],
    )
    def k(dst_hbm, deg_hbm, idx_v, one_v, zbuf, deg_sh):
        cid = lax.axis_index("c")
        sid = lax.axis_index("s")
        wid = sid * NC + cid
        _zero_fill(zbuf, RT, w)

        def ob(i, _):
            one_v[i, pl.ds(0, 16)] = jnp.ones((16,), _f32)
            return 0
        lax.fori_loop(0, CH, ob, 0)

        pltpu.sync_copy(zbuf, deg_sh.at[pl.ds(sid * RT, RT)])
        plsc.subcore_barrier()
        pltpu.sync_copy(dst_hbm.at[wid], idx_v)

        def body(j, _):
            pltpu.sync_copy(one_v, deg_sh.at[idx_v.at[j]], add=True)
            return 0

        lax.fori_loop(0, ROWS_W, body, 0)
        plsc.subcore_barrier()
        pltpu.sync_copy(deg_sh.at[pl.ds(sid * RT, RT)], zbuf)
        pltpu.sync_copy(zbuf, deg_hbm.at[cid, pl.ds(sid * RT, RT)])

    return k(dst3)


def kernel(x, edge_attr, edge_index, batch, target_index, target_class, params):
    p = params
    src = edge_index[0].astype(jnp.int32)
    dst = edge_index[1].astype(jnp.int32)
    def _padb(a, fill):
        return jnp.pad(a.reshape(10, E // 10), ((0, 0), (0, (EP - E) // 10)),
                       constant_values=fill).reshape(NW, ROWS_W, CH)
    src3 = _padb(src, 0)
    dst3 = _padb(dst, N)

    out = _node_pre(x, p)
    h = out
    e1p, w2f, c2f = _edge_pre(edge_attr, p)
    mmat = w2f.reshape(D, D, D).transpose(1, 0, 2).reshape(D * D, D)
    cmat = c2f.reshape(D, D)

    degs = _sc_degree(dst3)
    d0 = degs[0, :N, 0:1]
    d1 = degs[1, :N, 0:1]

    for _ in range(STEPS):
        xjp = _sc_gather(out, src3, D, ROWS_W)
        msgp = _msg(xjp, e1p, mmat, cmat)
        aggs = _sc_scatter(msgp, dst3)
        out = _gru(out, h, aggs[0, :N], aggs[1, :N], d0, d1, p)
        h = out

    batch2d = batch.astype(jnp.int32).reshape(N, 1)
    tab = _s2s(out, batch2d, p)

    atoms = jnp.concatenate([target_index[0], target_index[1]]).astype(jnp.int32)
    atoms3 = atoms.reshape(NW, (2 * T) // (NW * CH), CH)
    gath = _sc_gather(tab, atoms3, 3 * D, (2 * T) // (NW * CH))

    tc2d = target_class.astype(jnp.int32).reshape(T, 1)
    pred = _readout(gath[:T], gath[T:], tc2d, p)
    return pred.reshape(T)
